# baseline (speedup 1.0000x reference)
import jax
import jax.numpy as jnp
from jax.experimental import pallas as pl

B = 16384
K = 8192
DIM = 256
SEQ = 80

BM_ENC = 512
BM_DEC = 512


def _dec_body(state_ref, zq_ref, w4_ref, b4_ref, w5_ref, b5_ref,
              w6_ref, b6_ref, out_ref, zq_out_ref):
    dec = jnp.concatenate([state_ref[...], zq_ref[...]], axis=1)
    g = jnp.maximum(
        jnp.dot(dec, w4_ref[...], preferred_element_type=jnp.float32)
        + b4_ref[...], 0.0)
    g = jnp.maximum(
        jnp.dot(g, w5_ref[...], preferred_element_type=jnp.float32)
        + b5_ref[...], 0.0)
    out_ref[...] = (jnp.dot(g, w6_ref[...], preferred_element_type=jnp.float32)
                    + b6_ref[...])
    zq_out_ref[...] = zq_ref[...]


def _decode(state, z_q, w4, b4, w5, b5, w6, b6):
    grid = (B // BM_DEC,)
    return pl.pallas_call(
        _dec_body,
        grid=grid,
        in_specs=[
            pl.BlockSpec((BM_DEC, DIM), lambda i: (i, 0)),
            pl.BlockSpec((BM_DEC, DIM), lambda i: (i, 0)),
            pl.BlockSpec((2 * DIM, DIM), lambda i: (0, 0)),
            pl.BlockSpec((1, DIM), lambda i: (0, 0)),
            pl.BlockSpec((DIM, DIM), lambda i: (0, 0)),
            pl.BlockSpec((1, DIM), lambda i: (0, 0)),
            pl.BlockSpec((DIM, SEQ), lambda i: (0, 0)),
            pl.BlockSpec((1, SEQ), lambda i: (0, 0)),
        ],
        out_specs=[
            pl.BlockSpec((BM_DEC, SEQ), lambda i: (i, 0)),
            pl.BlockSpec((BM_DEC, DIM), lambda i: (i, 0)),
        ],
        out_shape=[
            jax.ShapeDtypeStruct((B, SEQ), jnp.float32),
            jax.ShapeDtypeStruct((B, DIM), jnp.float32),
        ],
    )(state, z_q, w4, b4, w5, b5, w6, b6)


def kernel(state, x, W1, b1, W2, b2, W3, b3, codebook, W4, b4, W5, b5, W6, b6):
    inp = jnp.concatenate([state, x], axis=1)
    h = jax.nn.relu(inp @ W1 + b1)
    h = jax.nn.relu(h @ W2 + b2)
    z_e_chain = h @ W3 + b3
    cb = jax.lax.stop_gradient(codebook)
    d = (jnp.sum(z_e_chain ** 2, axis=1, keepdims=True)
         - 2.0 * (z_e_chain @ cb.T)
         + jnp.sum(cb ** 2, axis=1)[None, :])
    indices = jnp.argmin(d, axis=1)
    z_q = jnp.take(codebook, indices, axis=0)

    b4r = b4.reshape(1, DIM)
    b5r = b5.reshape(1, DIM)
    b6r = b6.reshape(1, SEQ)

    x_tilde, z_q_out = _decode(state, z_q, W4, b4r, W5, b5r, W6, b6r)
    return (x_tilde, z_e_chain[:, :, None, None], z_q_out[:, :, None, None],
            indices)


# speedup vs baseline: 1.0046x; 1.0046x over previous
import jax
import jax.numpy as jnp
from jax.experimental import pallas as pl

B = 16384
K = 8192
DIM = 256
SEQ = 80

BM_ENC = 512
BM_DEC = 512


def _dec_body(state_ref, zq_ref, w4_ref, b4_ref, w5_ref, b5_ref,
              w6_ref, b6_ref, out_ref):
    dec = jnp.concatenate([state_ref[...], zq_ref[...]], axis=1)
    g = jnp.maximum(
        jnp.dot(dec, w4_ref[...], preferred_element_type=jnp.float32)
        + b4_ref[...], 0.0)
    g = jnp.maximum(
        jnp.dot(g, w5_ref[...], preferred_element_type=jnp.float32)
        + b5_ref[...], 0.0)
    out_ref[...] = (jnp.dot(g, w6_ref[...], preferred_element_type=jnp.float32)
                    + b6_ref[...])


def _decode(state, z_q, w4, b4, w5, b5, w6, b6):
    grid = (B // BM_DEC,)
    return pl.pallas_call(
        _dec_body,
        grid=grid,
        in_specs=[
            pl.BlockSpec((BM_DEC, DIM), lambda i: (i, 0)),
            pl.BlockSpec((BM_DEC, DIM), lambda i: (i, 0)),
            pl.BlockSpec((2 * DIM, DIM), lambda i: (0, 0)),
            pl.BlockSpec((1, DIM), lambda i: (0, 0)),
            pl.BlockSpec((DIM, DIM), lambda i: (0, 0)),
            pl.BlockSpec((1, DIM), lambda i: (0, 0)),
            pl.BlockSpec((DIM, SEQ), lambda i: (0, 0)),
            pl.BlockSpec((1, SEQ), lambda i: (0, 0)),
        ],
        out_specs=pl.BlockSpec((BM_DEC, SEQ), lambda i: (i, 0)),
        out_shape=jax.ShapeDtypeStruct((B, SEQ), jnp.float32),
    )(state, z_q, w4, b4, w5, b5, w6, b6)


def kernel(state, x, W1, b1, W2, b2, W3, b3, codebook, W4, b4, W5, b5, W6, b6):
    inp = jnp.concatenate([state, x], axis=1)
    h = jax.nn.relu(inp @ W1 + b1)
    h = jax.nn.relu(h @ W2 + b2)
    z_e_chain = h @ W3 + b3
    cb = jax.lax.stop_gradient(codebook)
    d = (jnp.sum(z_e_chain ** 2, axis=1, keepdims=True)
         - 2.0 * (z_e_chain @ cb.T)
         + jnp.sum(cb ** 2, axis=1)[None, :])
    indices = jnp.argmin(d, axis=1)
    z_q = jnp.take(codebook, indices, axis=0)

    b4r = b4.reshape(1, DIM)
    b5r = b5.reshape(1, DIM)
    b6r = b6.reshape(1, SEQ)

    x_tilde = _decode(state, z_q, W4, b4r, W5, b5r, W6, b6r)
    return (x_tilde, z_e_chain[:, :, None, None], z_q[:, :, None, None],
            indices)


# speedup vs baseline: 1.0330x; 1.0283x over previous
import jax
import jax.numpy as jnp
from jax.experimental import pallas as pl

B = 16384
K = 8192
DIM = 256
SEQ = 80

BM_ENC = 512
BM_DEC = 2048


def _dec_body(state_ref, zq_ref, w4_ref, b4_ref, w5_ref, b5_ref,
              w6_ref, b6_ref, out_ref):
    dec = jnp.concatenate([state_ref[...], zq_ref[...]], axis=1)
    g = jnp.maximum(
        jnp.dot(dec, w4_ref[...], preferred_element_type=jnp.float32)
        + b4_ref[...], 0.0)
    g = jnp.maximum(
        jnp.dot(g, w5_ref[...], preferred_element_type=jnp.float32)
        + b5_ref[...], 0.0)
    out_ref[...] = (jnp.dot(g, w6_ref[...], preferred_element_type=jnp.float32)
                    + b6_ref[...])


def _decode(state, z_q, w4, b4, w5, b5, w6, b6):
    grid = (B // BM_DEC,)
    return pl.pallas_call(
        _dec_body,
        grid=grid,
        in_specs=[
            pl.BlockSpec((BM_DEC, DIM), lambda i: (i, 0)),
            pl.BlockSpec((BM_DEC, DIM), lambda i: (i, 0)),
            pl.BlockSpec((2 * DIM, DIM), lambda i: (0, 0)),
            pl.BlockSpec((1, DIM), lambda i: (0, 0)),
            pl.BlockSpec((DIM, DIM), lambda i: (0, 0)),
            pl.BlockSpec((1, DIM), lambda i: (0, 0)),
            pl.BlockSpec((DIM, SEQ), lambda i: (0, 0)),
            pl.BlockSpec((1, SEQ), lambda i: (0, 0)),
        ],
        out_specs=pl.BlockSpec((BM_DEC, SEQ), lambda i: (i, 0)),
        out_shape=jax.ShapeDtypeStruct((B, SEQ), jnp.float32),
    )(state, z_q, w4, b4, w5, b5, w6, b6)


def kernel(state, x, W1, b1, W2, b2, W3, b3, codebook, W4, b4, W5, b5, W6, b6):
    inp = jnp.concatenate([state, x], axis=1)
    h = jax.nn.relu(inp @ W1 + b1)
    h = jax.nn.relu(h @ W2 + b2)
    z_e_chain = h @ W3 + b3
    cb = jax.lax.stop_gradient(codebook)
    d = (jnp.sum(z_e_chain ** 2, axis=1, keepdims=True)
         - 2.0 * (z_e_chain @ cb.T)
         + jnp.sum(cb ** 2, axis=1)[None, :])
    indices = jnp.argmin(d, axis=1)
    z_q = jnp.take(codebook, indices, axis=0)

    b4r = b4.reshape(1, DIM)
    b5r = b5.reshape(1, DIM)
    b6r = b6.reshape(1, SEQ)

    x_tilde = _decode(state, z_q, W4, b4r, W5, b5r, W6, b6r)
    return (x_tilde, z_e_chain[:, :, None, None], z_q[:, :, None, None],
            indices)


# speedup vs baseline: 1.0336x; 1.0006x over previous
import jax
import jax.numpy as jnp
from jax.experimental import pallas as pl

B = 16384
K = 8192
DIM = 256
SEQ = 80

BM_ENC = 512
BM_DEC = 4096


def _dec_body(state_ref, zq_ref, w4_ref, b4_ref, w5_ref, b5_ref,
              w6_ref, b6_ref, out_ref):
    dec = jnp.concatenate([state_ref[...], zq_ref[...]], axis=1)
    g = jnp.maximum(
        jnp.dot(dec, w4_ref[...], preferred_element_type=jnp.float32)
        + b4_ref[...], 0.0)
    g = jnp.maximum(
        jnp.dot(g, w5_ref[...], preferred_element_type=jnp.float32)
        + b5_ref[...], 0.0)
    out_ref[...] = (jnp.dot(g, w6_ref[...], preferred_element_type=jnp.float32)
                    + b6_ref[...])


def _decode(state, z_q, w4, b4, w5, b5, w6, b6):
    grid = (B // BM_DEC,)
    return pl.pallas_call(
        _dec_body,
        grid=grid,
        in_specs=[
            pl.BlockSpec((BM_DEC, DIM), lambda i: (i, 0)),
            pl.BlockSpec((BM_DEC, DIM), lambda i: (i, 0)),
            pl.BlockSpec((2 * DIM, DIM), lambda i: (0, 0)),
            pl.BlockSpec((1, DIM), lambda i: (0, 0)),
            pl.BlockSpec((DIM, DIM), lambda i: (0, 0)),
            pl.BlockSpec((1, DIM), lambda i: (0, 0)),
            pl.BlockSpec((DIM, SEQ), lambda i: (0, 0)),
            pl.BlockSpec((1, SEQ), lambda i: (0, 0)),
        ],
        out_specs=pl.BlockSpec((BM_DEC, SEQ), lambda i: (i, 0)),
        out_shape=jax.ShapeDtypeStruct((B, SEQ), jnp.float32),
    )(state, z_q, w4, b4, w5, b5, w6, b6)


def kernel(state, x, W1, b1, W2, b2, W3, b3, codebook, W4, b4, W5, b5, W6, b6):
    inp = jnp.concatenate([state, x], axis=1)
    h = jax.nn.relu(inp @ W1 + b1)
    h = jax.nn.relu(h @ W2 + b2)
    z_e_chain = h @ W3 + b3
    cb = jax.lax.stop_gradient(codebook)
    d = (jnp.sum(z_e_chain ** 2, axis=1, keepdims=True)
         - 2.0 * (z_e_chain @ cb.T)
         + jnp.sum(cb ** 2, axis=1)[None, :])
    indices = jnp.argmin(d, axis=1)
    z_q = jnp.take(codebook, indices, axis=0)

    b4r = b4.reshape(1, DIM)
    b5r = b5.reshape(1, DIM)
    b6r = b6.reshape(1, SEQ)

    x_tilde = _decode(state, z_q, W4, b4r, W5, b5r, W6, b6r)
    return (x_tilde, z_e_chain[:, :, None, None], z_q[:, :, None, None],
            indices)


# speedup vs baseline: 1.0337x; 1.0001x over previous
import jax
import jax.numpy as jnp
from jax.experimental import pallas as pl

B = 16384
K = 8192
DIM = 256
SEQ = 80

BM_DEC = 4096


def _dec_body(state_ref, zq_ref, w4_ref, b4_ref, w5_ref, b5_ref,
              w6_ref, b6_ref, out_ref):
    dec = jnp.concatenate([state_ref[...], zq_ref[...]], axis=1)
    g = jnp.maximum(
        jnp.dot(dec, w4_ref[...], preferred_element_type=jnp.float32)
        + b4_ref[...], 0.0)
    g = jnp.maximum(
        jnp.dot(g, w5_ref[...], preferred_element_type=jnp.float32)
        + b5_ref[...], 0.0)
    out_ref[...] = (jnp.dot(g, w6_ref[...], preferred_element_type=jnp.float32)
                    + b6_ref[...])


def _decode(state, z_q, w4, b4, w5, b5, w6, b6):
    grid = (B // BM_DEC,)
    return pl.pallas_call(
        _dec_body,
        grid=grid,
        in_specs=[
            pl.BlockSpec((BM_DEC, DIM), lambda i: (i, 0)),
            pl.BlockSpec((BM_DEC, DIM), lambda i: (i, 0)),
            pl.BlockSpec((2 * DIM, DIM), lambda i: (0, 0)),
            pl.BlockSpec((1, DIM), lambda i: (0, 0)),
            pl.BlockSpec((DIM, DIM), lambda i: (0, 0)),
            pl.BlockSpec((1, DIM), lambda i: (0, 0)),
            pl.BlockSpec((DIM, SEQ), lambda i: (0, 0)),
            pl.BlockSpec((1, SEQ), lambda i: (0, 0)),
        ],
        out_specs=pl.BlockSpec((BM_DEC, SEQ), lambda i: (i, 0)),
        out_shape=jax.ShapeDtypeStruct((B, SEQ), jnp.float32),
    )(state, z_q, w4, b4, w5, b5, w6, b6)


def kernel(state, x, W1, b1, W2, b2, W3, b3, codebook, W4, b4, W5, b5, W6, b6):
    inp = jnp.concatenate([state, x], axis=1)
    h = jax.nn.relu(inp @ W1 + b1)
    h = jax.nn.relu(h @ W2 + b2)
    z_e_chain = h @ W3 + b3
    cb = jax.lax.stop_gradient(codebook)
    d = (jnp.sum(z_e_chain ** 2, axis=1, keepdims=True)
         - 2.0 * (z_e_chain @ cb.T)
         + jnp.sum(cb ** 2, axis=1)[None, :])
    indices = jnp.argmin(d, axis=1)
    z_q = jnp.take(codebook, indices, axis=0)

    b4r = b4.reshape(1, DIM)
    b5r = b5.reshape(1, DIM)
    b6r = b6.reshape(1, SEQ)

    x_tilde = _decode(state, z_q, W4, b4r, W5, b5r, W6, b6r)
    return (x_tilde, z_e_chain[:, :, None, None], z_q[:, :, None, None],
            indices)
